# Initial kernel scaffold; baseline (speedup 1.0000x reference)
import jax
import jax.numpy as jnp
from jax.experimental import pallas as pl


def kernel(moving_volume, lh_white_vertices, lh_faces, lh_thickness, vox2ras_tkr, transform_params):
    raise NotImplementedError("write your pallas kernel here")


# speedup vs baseline: 37.1938x; 37.1938x over previous
import functools

import jax
import jax.numpy as jnp
from jax import lax
from jax.experimental import pallas as pl
from jax.experimental.pallas import tpu as pltpu
from jax.experimental.pallas import tpu_sc as plsc

NC, NS, L = 2, 16, 16
NW = NC * NS

N_VERT = 150000
N_FACE = 300000
VOL = 256

CH = 128
V_CHUNKS = -(-N_VERT // (NW * CH))
NVP = NW * CH * V_CHUNKS
VPW = CH * V_CHUNKS
F_CHUNKS = -(-N_FACE // (NW * CH))
NFP = NW * CH * F_CHUNKS
ROWS_PER_TILE = NVP // NS

_mesh = plsc.VectorSubcoreMesh(core_axis_name="c", subcore_axis_name="s")


def _splat_i(v):
    return jnp.full((L,), v, jnp.int32)


def _rsqrt_nr(x):
    i = lax.bitcast_convert_type(x, jnp.int32)
    i = 0x5F3759DF - lax.shift_right_logical(i, 1)
    y = lax.bitcast_convert_type(i, jnp.float32)
    for _ in range(3):
        y = y * (1.5 - 0.5 * x * y * y)
    return y


@functools.partial(
    pl.kernel,
    out_type=(jax.ShapeDtypeStruct((NC, 1, NVP), jnp.float32),
              jax.ShapeDtypeStruct((NC, 1, NVP), jnp.float32),
              jax.ShapeDtypeStruct((NC, 1, NVP), jnp.float32)),
    mesh=_mesh,
    scratch_types=[
        pltpu.VMEM_SHARED((NVP,), jnp.float32),
        pltpu.VMEM_SHARED((NVP,), jnp.float32),
        pltpu.VMEM_SHARED((NVP,), jnp.float32),
        pltpu.VMEM((F_CHUNKS, CH), jnp.int32),
        pltpu.VMEM((F_CHUNKS, CH), jnp.int32),
        pltpu.VMEM((F_CHUNKS, CH), jnp.int32),
        pltpu.VMEM((3, CH), jnp.float32),
        pltpu.VMEM((3, CH), jnp.float32),
        pltpu.VMEM((3, CH), jnp.float32),
        pltpu.VMEM((CH,), jnp.float32),
        pltpu.VMEM((CH,), jnp.float32),
        pltpu.VMEM((CH,), jnp.float32),
        pltpu.SemaphoreType.DMA,
    ],
)
def _normals_kernel(vx_hbm, vy_hbm, vz_hbm, f0_hbm, f1_hbm, f2_hbm,
                    zeros_hbm, ox_hbm, oy_hbm, oz_hbm,
                    accx, accy, accz, f0b, f1b, f2b,
                    c0b, c1b, c2b, fnxb, fnyb, fnzb, sem):
    c = lax.axis_index("c")
    s = lax.axis_index("s")
    wid = s * NC + c

    r0 = s * ROWS_PER_TILE
    sl_out = pl.ds(r0, ROWS_PER_TILE)
    pltpu.sync_copy(zeros_hbm.at[sl_out], accx.at[sl_out])
    pltpu.sync_copy(zeros_hbm.at[sl_out], accy.at[sl_out])
    pltpu.sync_copy(zeros_hbm.at[sl_out], accz.at[sl_out])
    pltpu.sync_copy(f0_hbm.at[wid], f0b)
    pltpu.sync_copy(f1_hbm.at[wid], f1b)
    pltpu.sync_copy(f2_hbm.at[wid], f2b)
    plsc.subcore_barrier()

    def chunk_body(j, carry):
        i0 = f0b.at[j]
        i1 = f1b.at[j]
        i2 = f2b.at[j]
        descs = []
        for cb, idx in ((c0b, i0), (c1b, i1), (c2b, i2)):
            descs.append(pltpu.async_copy(vx_hbm.at[idx], cb.at[0], sem))
            descs.append(pltpu.async_copy(vy_hbm.at[idx], cb.at[1], sem))
            descs.append(pltpu.async_copy(vz_hbm.at[idx], cb.at[2], sem))
        for d in descs:
            d.wait()

        def sub(s8, t):
            sl = pl.ds(s8 * L, L)
            v0x = c0b[0, sl]
            v0y = c0b[1, sl]
            v0z = c0b[2, sl]
            ax = c1b[0, sl] - v0x
            ay = c1b[1, sl] - v0y
            az = c1b[2, sl] - v0z
            bx = c2b[0, sl] - v0x
            by = c2b[1, sl] - v0y
            bz = c2b[2, sl] - v0z
            fnxb[sl] = ay * bz - az * by
            fnyb[sl] = az * bx - ax * bz
            fnzb[sl] = ax * by - ay * bx
            return t

        lax.fori_loop(0, CH // L, sub, 0)
        for idx in (i0, i1, i2):
            pltpu.sync_copy(fnxb, accx.at[idx], add=True)
            pltpu.sync_copy(fnyb, accy.at[idx], add=True)
            pltpu.sync_copy(fnzb, accz.at[idx], add=True)
        return carry

    lax.fori_loop(0, F_CHUNKS, chunk_body, 0)
    plsc.subcore_barrier()
    pltpu.sync_copy(accx.at[sl_out], ox_hbm.at[c, 0, sl_out])
    pltpu.sync_copy(accy.at[sl_out], oy_hbm.at[c, 0, sl_out])
    pltpu.sync_copy(accz.at[sl_out], oz_hbm.at[c, 0, sl_out])


@functools.partial(
    pl.kernel,
    out_type=jax.ShapeDtypeStruct((NW, L), jnp.float32),
    mesh=_mesh,
    scratch_types=[
        pltpu.VMEM((VPW,), jnp.float32),
        pltpu.VMEM((VPW,), jnp.float32),
        pltpu.VMEM((VPW,), jnp.float32),
        pltpu.VMEM((VPW,), jnp.float32),
        pltpu.VMEM((VPW,), jnp.float32),
        pltpu.VMEM((VPW,), jnp.float32),
        pltpu.VMEM((VPW,), jnp.float32),
        pltpu.VMEM((VPW,), jnp.float32),
        pltpu.VMEM((VPW,), jnp.float32),
        pltpu.VMEM((VPW,), jnp.float32),
        pltpu.VMEM((CH,), jnp.float32),
        pltpu.VMEM((16, CH), jnp.int32),
        pltpu.VMEM((8, CH), jnp.float32),
        pltpu.VMEM((16, CH), jnp.float32),
        pltpu.VMEM((L,), jnp.float32),
        pltpu.SemaphoreType.DMA,
    ],
)
def _sample_kernel(vx_hbm, vy_hbm, vz_hbm, th_hbm,
                   nx_hbm, ny_hbm, nz_hbm, vol_hbm, m_hbm, out_hbm,
                   vxb, vyb, vzb, n0xb, n0yb, n0zb, n1xb, n1yb, n1zb,
                   thb, mb, idxb, frb, cvb, accb, sem):
    c = lax.axis_index("c")
    s = lax.axis_index("s")
    wid = s * NC + c
    base = wid * VPW
    sl_in = pl.ds(base, VPW)

    pltpu.sync_copy(vx_hbm.at[sl_in], vxb)
    pltpu.sync_copy(vy_hbm.at[sl_in], vyb)
    pltpu.sync_copy(vz_hbm.at[sl_in], vzb)
    pltpu.sync_copy(th_hbm.at[sl_in], thb)
    pltpu.sync_copy(nx_hbm.at[0, 0, sl_in], n0xb)
    pltpu.sync_copy(ny_hbm.at[0, 0, sl_in], n0yb)
    pltpu.sync_copy(nz_hbm.at[0, 0, sl_in], n0zb)
    pltpu.sync_copy(nx_hbm.at[1, 0, sl_in], n1xb)
    pltpu.sync_copy(ny_hbm.at[1, 0, sl_in], n1yb)
    pltpu.sync_copy(nz_hbm.at[1, 0, sl_in], n1zb)
    pltpu.sync_copy(m_hbm, mb)

    mrow = mb[pl.ds(0, L)]
    m = [mrow[k] for k in range(12)]
    hi = jnp.float32(VOL - 1 - 1e-4)
    iota = lax.iota(jnp.int32, L)

    def chunk(j, acc):
        def p1(s8, t):
            o = j * CH + s8 * L
            sl = pl.ds(o, L)
            col = pl.ds(s8 * L, L)
            vx = vxb[sl]
            vy = vyb[sl]
            vz = vzb[sl]
            nx = n0xb[sl] + n1xb[sl]
            ny = n0yb[sl] + n1yb[sl]
            nz = n0zb[sl] + n1zb[sl]
            th = thb[sl]
            ss = nx * nx + ny * ny + nz * nz
            inv = 1.0 / (ss * _rsqrt_nr(ss) + 1e-8)
            ux = nx * inv
            uy = ny * inv
            uz = nz * inv
            for surf in range(2):
                f = jnp.float32(-2.0) if surf == 0 else 0.5 * th
                px = vx + f * ux
                py = vy + f * uy
                pz = vz + f * uz
                tx = m[0] * px + m[1] * py + m[2] * pz + m[3]
                ty = m[4] * px + m[5] * py + m[6] * pz + m[7]
                tz = m[8] * px + m[9] * py + m[10] * pz + m[11]
                tx = jnp.minimum(jnp.maximum(tx, 0.0), hi)
                ty = jnp.minimum(jnp.maximum(ty, 0.0), hi)
                tz = jnp.minimum(jnp.maximum(tz, 0.0), hi)
                x0 = tx.astype(jnp.int32)
                y0 = ty.astype(jnp.int32)
                z0 = tz.astype(jnp.int32)
                b000 = x0 * (VOL * VOL) + y0 * VOL + z0
                k0 = surf * 8
                idxb[k0 + 0, col] = b000
                idxb[k0 + 1, col] = b000 + 1
                idxb[k0 + 2, col] = b000 + VOL
                idxb[k0 + 3, col] = b000 + VOL + 1
                idxb[k0 + 4, col] = b000 + VOL * VOL
                idxb[k0 + 5, col] = b000 + VOL * VOL + 1
                idxb[k0 + 6, col] = b000 + VOL * VOL + VOL
                idxb[k0 + 7, col] = b000 + VOL * VOL + VOL + 1
                f0 = surf * 3
                frb[f0 + 0, col] = tx - x0.astype(jnp.float32)
                frb[f0 + 1, col] = ty - y0.astype(jnp.float32)
                frb[f0 + 2, col] = tz - z0.astype(jnp.float32)
            return t

        lax.fori_loop(0, CH // L, p1, 0)

        descs = [pltpu.async_copy(vol_hbm.at[idxb.at[k]], cvb.at[k], sem)
                 for k in range(16)]
        for d in descs:
            d.wait()

        def p2(s8, a):
            col = pl.ds(s8 * L, L)
            vals = []
            for surf in range(2):
                k0 = surf * 8
                f0 = surf * 3
                fx = frb[f0 + 0, col]
                fy = frb[f0 + 1, col]
                fz = frb[f0 + 2, col]
                gx = 1.0 - fx
                gy = 1.0 - fy
                gz = 1.0 - fz
                v = (cvb[k0 + 0, col] * gx * gy * gz
                     + cvb[k0 + 1, col] * gx * gy * fz
                     + cvb[k0 + 2, col] * gx * fy * gz
                     + cvb[k0 + 3, col] * gx * fy * fz
                     + cvb[k0 + 4, col] * fx * gy * gz
                     + cvb[k0 + 5, col] * fx * gy * fz
                     + cvb[k0 + 6, col] * fx * fy * gz
                     + cvb[k0 + 7, col] * fx * fy * fz)
                vals.append(v)
            vwm, vgm = vals
            q = 100.0 * (vgm - vwm) / (0.5 * (vgm + vwm) + 1e-6)
            term = 2.0 - 2.0 / (jnp.exp(q) + 1.0)
            g = base + j * CH + s8 * L + iota
            return a + jnp.where(g < N_VERT, term, jnp.float32(0.0))

        return lax.fori_loop(0, CH // L, p2, acc)

    acc = lax.fori_loop(0, V_CHUNKS, chunk, jnp.zeros((L,), jnp.float32))
    accb[...] = acc
    pltpu.sync_copy(accb, out_hbm.at[wid])


def kernel(moving_volume, lh_white_vertices, lh_faces, lh_thickness,
           vox2ras_tkr, transform_params):
    p = transform_params
    cos = jnp.cos(p[3:6])
    sin = jnp.sin(p[3:6])
    one = jnp.ones((), p.dtype)
    zero = jnp.zeros((), p.dtype)
    rx = jnp.stack([one, zero, zero, zero, cos[0], -sin[0], zero, sin[0],
                    cos[0]]).reshape(3, 3)
    ry = jnp.stack([cos[1], zero, sin[1], zero, one, zero, -sin[1], zero,
                    cos[1]]).reshape(3, 3)
    rz = jnp.stack([cos[2], -sin[2], zero, sin[2], cos[2], zero, zero, zero,
                    one]).reshape(3, 3)
    rot = rx @ ry @ rz
    top = jnp.concatenate([rot, p[:3][:, None]], axis=1)
    bottom = jnp.array([[0.0, 0.0, 0.0, 1.0]], dtype=p.dtype)
    t4 = jnp.concatenate([top, bottom], axis=0)
    m4 = jnp.linalg.inv(vox2ras_tkr) @ t4
    mvec = jnp.concatenate([m4[:3, :].reshape(12),
                            jnp.zeros((CH - 12,), jnp.float32)])

    vpad = jnp.full((NVP - N_VERT,), 100.0, jnp.float32)
    vx = jnp.concatenate([lh_white_vertices[:, 0], vpad])
    vy = jnp.concatenate([lh_white_vertices[:, 1], vpad])
    vz = jnp.concatenate([lh_white_vertices[:, 2], vpad])
    th_p = jnp.concatenate(
        [lh_thickness, jnp.ones((NVP - N_VERT,), jnp.float32)])
    fpad = jnp.zeros((NFP - N_FACE,), jnp.int32)
    f0 = jnp.concatenate([lh_faces[:, 0], fpad]).reshape(NW, F_CHUNKS, CH)
    f1 = jnp.concatenate([lh_faces[:, 1], fpad]).reshape(NW, F_CHUNKS, CH)
    f2 = jnp.concatenate([lh_faces[:, 2], fpad]).reshape(NW, F_CHUNKS, CH)
    zeros1 = jnp.zeros((NVP,), jnp.float32)
    vol_flat = moving_volume.reshape(-1)

    nxp, nyp, nzp = _normals_kernel(vx, vy, vz, f0, f1, f2, zeros1)
    partials = _sample_kernel(vx, vy, vz, th_p, nxp, nyp, nzp,
                              vol_flat, mvec)
    return jnp.sum(partials) / jnp.float32(N_VERT)
